# Initial kernel scaffold; baseline (speedup 1.0000x reference)
import jax
import jax.numpy as jnp
from jax.experimental import pallas as pl


def kernel(feature, label, centers):
    raise NotImplementedError("write your pallas kernel here")


# speedup vs baseline: 2.2052x; 2.2052x over previous
import functools

import jax
import jax.numpy as jnp
from jax import lax
from jax.experimental import pallas as pl
from jax.experimental.pallas import tpu as pltpu
from jax.experimental.pallas import tpu_sc as plsc

_NUM_CLASSES = 10
_FEAT = 2
_BATCH = 16384

_NC = 2
_NS = 16
_LANES = 16
_NW = _NC * _NS
_B_PER_W = _BATCH // _NW
_STEPS = _B_PER_W // _LANES


def _reg_gather(src, idx):
    return src.at[idx].get(mode="promise_in_bounds")


def _body(feat_hbm, lab_hbm, cent_hbm, out_hbm,
          feat_v, lab_v, cent_v, acc_v, tot_v, shared):
    cid = lax.axis_index("c")
    sid = lax.axis_index("s")
    wid = sid * _NC + cid
    base = wid * _B_PER_W

    pltpu.sync_copy(feat_hbm.at[pl.ds(base * _FEAT, _B_PER_W * _FEAT)], feat_v)
    pltpu.sync_copy(lab_hbm.at[pl.ds(base, _B_PER_W)], lab_v)
    pltpu.sync_copy(cent_hbm, cent_v)

    cent_x = cent_v[pl.ds(0, _LANES)]
    cent_y = cent_v[pl.ds(_LANES, _LANES)]

    lane = lax.iota(jnp.int32, _LANES)
    half_a = lane >> 1
    half_b = half_a + 8
    is_y = (lane & 1) == 1

    acc = jnp.zeros((_LANES,), jnp.float32)
    for i in range(_STEPS):
        labs = lab_v[pl.ds(i * _LANES, _LANES)]
        for j, half in enumerate((half_a, half_b)):
            lab8 = _reg_gather(labs, half)
            cx = _reg_gather(cent_x, lab8)
            cy = _reg_gather(cent_y, lab8)
            c = jnp.where(is_y, cy, cx)
            f = feat_v[pl.ds(i * 2 * _LANES + j * _LANES, _LANES)]
            d = f - c
            acc = acc + d * d

    acc_v[...] = acc
    pltpu.sync_copy(acc_v, out_hbm.at[wid])


_sc_center_loss = functools.partial(
    pl.kernel,
    out_type=jax.ShapeDtypeStruct((_NW, _LANES), jnp.float32),
    mesh=plsc.VectorSubcoreMesh(core_axis_name="c", subcore_axis_name="s"),
    scratch_types=[
        pltpu.VMEM((_B_PER_W * _FEAT,), jnp.float32),
        pltpu.VMEM((_B_PER_W,), jnp.int32),
        pltpu.VMEM((2 * _LANES,), jnp.float32),
        pltpu.VMEM((_LANES,), jnp.float32),
        pltpu.VMEM((_NS, _LANES), jnp.float32),
        pltpu.VMEM_SHARED((_NS, _LANES), jnp.float32),
    ],
)(_body)


@jax.jit
def kernel(feature, label, centers):
    feat_flat = feature.reshape(-1)
    pad = jnp.zeros((_LANES - _NUM_CLASSES,), jnp.float32)
    cent_cols = jnp.concatenate(
        [centers[:, 0], pad, centers[:, 1], pad])
    out = _sc_center_loss(feat_flat, label, cent_cols)
    return jnp.sum(out) * (1.0 / _BATCH)
